# baseline (speedup 1.0000x reference)
import functools

import jax
import jax.numpy as jnp
from jax import lax
from jax.experimental import pallas as pl
from jax.experimental.pallas import tpu as pltpu
from jax.experimental.pallas import tpu_sc as plsc

_W_ENERGY = 1.0
_W_FORCE = 0.999

_NC, _NS, _L = 2, 16, 16
_NW = _NC * _NS
_N = 131072
_M = 4096
_APW = _N // _NW
_APS = _N // _NS
_MPW = _M // _NW

_mesh = plsc.VectorSubcoreMesh(core_axis_name="c", subcore_axis_name="s")


@functools.partial(
    pl.kernel,
    out_type=jax.ShapeDtypeStruct((_NC, _L), jnp.float32),
    mesh=_mesh,
    compiler_params=pltpu.CompilerParams(
        needs_layout_passes=False, use_tc_tiling_on_sc=False),
    scratch_types=[
        pltpu.VMEM((_APS,), jnp.int32),
        pltpu.VMEM((_APS,), jnp.float32),
        pltpu.VMEM((_APW * 3,), jnp.float32),
        pltpu.VMEM((_APW * 3,), jnp.float32),
        pltpu.VMEM((_M,), jnp.float32),
        pltpu.VMEM((_MPW,), jnp.float32),
        pltpu.VMEM((_MPW,), jnp.float32),
        pltpu.VMEM((_NS * _L,), jnp.float32),
        pltpu.VMEM((_L,), jnp.float32),
        pltpu.VMEM_SHARED((_M,), jnp.float32),
        pltpu.VMEM_SHARED((_NS * _L,), jnp.float32),
    ],
)
def _loss_sc(ep_h, et_h, fp_h, ft_h, idx_h, out_h,
             idx_v, ones_v, fp_v, ft_v, cnt_v, ep_v, et_v, red_v, tmp_v,
             counts_sh, parts_sh):
    cid = lax.axis_index("c")
    sid = lax.axis_index("s")
    w2 = sid * _NC + cid
    abase = w2 * _APW
    mbase = w2 * _MPW

    pltpu.sync_copy(idx_h.at[pl.ds(sid * _APS, _APS)], idx_v)
    pltpu.sync_copy(fp_h.at[pl.ds(abase * 3, _APW * 3)], fp_v)
    pltpu.sync_copy(ft_h.at[pl.ds(abase * 3, _APW * 3)], ft_v)
    pltpu.sync_copy(ep_h.at[pl.ds(mbase, _MPW)], ep_v)
    pltpu.sync_copy(et_h.at[pl.ds(mbase, _MPW)], et_v)

    ones16 = jnp.ones((_L,), jnp.float32)
    zeros16 = jnp.zeros((_L,), jnp.float32)
    iota = lax.iota(jnp.int32, _L)

    def _fill_ones(i, _):
        ones_v[pl.ds(i * _L, _L)] = ones16
        return 0
    lax.fori_loop(0, _APS // _L, _fill_ones, 0)

    def _fill_zero(i, _):
        red_v[pl.ds(i * _L, _L)] = zeros16
        return 0
    lax.fori_loop(0, _NS, _fill_zero, 0)

    pltpu.sync_copy(red_v, counts_sh.at[pl.ds(sid * (_M // _NS), _M // _NS)])
    plsc.subcore_barrier()
    pltpu.sync_copy(ones_v, counts_sh.at[idx_v], add=True)
    plsc.subcore_barrier()

    pltpu.sync_copy(counts_sh, cnt_v)

    def _invert(i, _):
        c = cnt_v[pl.ds(i * _L, _L)]
        cnt_v[pl.ds(i * _L, _L)] = ones16 / c
        return 0
    lax.fori_loop(0, _M // _L, _invert, 0)

    iota3 = iota * 3
    idx_off = cid * _APW

    def _force(i, acc):
        a0 = i * _L
        p0 = a0 * 3 + iota3
        iv = idx_v[pl.ds(idx_off + a0, _L)]
        icnt = plsc.load_gather(cnt_v, [iv])
        d0 = plsc.load_gather(fp_v, [p0]) - plsc.load_gather(ft_v, [p0])
        d1 = plsc.load_gather(fp_v, [p0 + 1]) - plsc.load_gather(ft_v, [p0 + 1])
        d2 = plsc.load_gather(fp_v, [p0 + 2]) - plsc.load_gather(ft_v, [p0 + 2])
        s = d0 * d0 + d1 * d1 + d2 * d2
        return acc + s * icnt
    f_acc = lax.fori_loop(0, _APW // _L, _force, jnp.zeros((_L,), jnp.float32))

    def _energy(j, acc):
        d = ep_v[pl.ds(j * _L, _L)] - et_v[pl.ds(j * _L, _L)]
        ic = cnt_v[pl.ds(mbase + j * _L, _L)]
        return acc + d * d * ic
    e_acc = lax.fori_loop(0, _MPW // _L, _energy, jnp.zeros((_L,), jnp.float32))

    e_part = jnp.sum(e_acc)
    f_part = jnp.sum(f_acc)
    pv = jnp.where(iota == 0, e_part, jnp.where(iota == 1, f_part, 0.0))
    tmp_v[...] = pv
    pltpu.sync_copy(tmp_v, parts_sh.at[pl.ds(sid * _L, _L)])
    plsc.subcore_barrier()

    @pl.when(sid == 0)
    def _finalize():
        pltpu.sync_copy(parts_sh, red_v)

        def _reduce(s, acc):
            return acc + red_v[pl.ds(s * _L, _L)]
        sums = lax.fori_loop(0, _NS, _reduce, jnp.zeros((_L,), jnp.float32))
        e_b = jnp.sum(jnp.where(iota == 0, sums, 0.0))
        f_b = jnp.sum(jnp.where(iota == 1, sums, 0.0))
        e_loss = e_b * (_W_ENERGY / _M)
        f_loss = f_b * (_W_FORCE / _M)
        tot = e_loss + f_loss
        outv = jnp.where(iota == 0, tot,
                         jnp.where(iota == 1, e_loss,
                                   jnp.where(iota == 2, f_loss, 0.0)))
        tmp_v[...] = outv
        pltpu.sync_copy(tmp_v, out_h.at[cid])


def kernel(per_molecule_energy_predict, per_molecule_energy_true,
           per_atom_force_predict, per_atom_force_true,
           atomic_subsystem_indices):
    out = _loss_sc(
        per_molecule_energy_predict.reshape(_M),
        per_molecule_energy_true.reshape(_M),
        per_atom_force_predict.reshape(_N * 3),
        per_atom_force_true.reshape(_N * 3),
        atomic_subsystem_indices.astype(jnp.int32),
    )
    total = out[0, 0] + out[1, 0]
    e_loss = out[0, 1] + out[1, 1]
    f_loss = out[0, 2] + out[1, 2]
    return (total, e_loss, f_loss)
